# baseline (speedup 1.0000x reference)
import functools

import jax
import jax.numpy as jnp
import numpy as np
from jax import lax
from jax.experimental import pallas as pl
from jax.experimental.pallas import tpu as pltpu
from jax.experimental.pallas import tpu_sc as plsc

_NUM_CLASSES = 1000
_SMOOTHING = 0.1
_BATCH = 16384

_NUM_WORKERS = 32
_ROWS_PER_WORKER = _BATCH // _NUM_WORKERS
_CHUNK = 32
_NCHUNKS = _ROWS_PER_WORKER // _CHUNK
_LANES = 16
_CHUNK_WORDS = _CHUNK * _NUM_CLASSES

_BASE = float(np.float32(_SMOOTHING / _NUM_CLASSES))
_PEAK = float(np.float32(np.float32(_BASE) + np.float32(1.0 - _SMOOTHING)))


def _sc_body(target_hbm, out_hbm, tgt_v, buf0, buf1, sem0, sem1):
    wid = lax.axis_index("s") * 2 + lax.axis_index("c")
    row0 = wid * _ROWS_PER_WORKER

    base_vec = jnp.full((_LANES,), _BASE, jnp.float32)
    peak_vec = jnp.full((_LANES,), _PEAK, jnp.float32)
    lane_iota = lax.broadcasted_iota(jnp.int32, (_LANES,), 0)

    pltpu.sync_copy(target_hbm.at[pl.ds(row0, _ROWS_PER_WORKER)], tgt_v)

    def fill_grp(j, _):
        buf0[pl.ds(j * _LANES, _LANES)] = base_vec
        buf1[pl.ds(j * _LANES, _LANES)] = base_vec
        return 0

    lax.fori_loop(0, _CHUNK_WORDS // _LANES, fill_grp, 0)

    bufs = (buf0, buf1)
    sems = (sem0, sem1)
    copies = [None, None]
    groups = _CHUNK // _LANES

    for c in range(_NCHUNKS):
        slot = c % 2
        buf = bufs[slot]
        if copies[slot] is not None:
            copies[slot].wait()
            for g in range(groups):
                pt = tgt_v[pl.ds((c - 2) * _CHUNK + g * _LANES, _LANES)]
                pidx = (lane_iota + g * _LANES) * _NUM_CLASSES + pt
                plsc.store_scatter(buf, [pidx], base_vec)
        for g in range(groups):
            t = tgt_v[pl.ds(c * _CHUNK + g * _LANES, _LANES)]
            idx = (lane_iota + g * _LANES) * _NUM_CLASSES + t
            plsc.store_scatter(buf, [idx], peak_vec)
        copies[slot] = pltpu.async_copy(
            buf,
            out_hbm.at[pl.ds((row0 + c * _CHUNK) * _NUM_CLASSES, _CHUNK_WORDS)],
            sems[slot])

    copies[(_NCHUNKS - 2) % 2].wait()
    copies[(_NCHUNKS - 1) % 2].wait()


@jax.jit
def _sc_call(target):
    mesh = plsc.VectorSubcoreMesh(core_axis_name="c", subcore_axis_name="s")
    flat = pl.kernel(
        _sc_body,
        mesh=mesh,
        compiler_params=pltpu.CompilerParams(needs_layout_passes=False),
        out_type=jax.ShapeDtypeStruct((_BATCH * _NUM_CLASSES,), jnp.float32),
        scratch_types=[
            pltpu.VMEM((_ROWS_PER_WORKER,), jnp.int32),
            pltpu.VMEM((_CHUNK_WORDS,), jnp.float32),
            pltpu.VMEM((_CHUNK_WORDS,), jnp.float32),
            pltpu.SemaphoreType.DMA,
            pltpu.SemaphoreType.DMA,
        ],
    )(target)
    return flat.reshape(_BATCH, _NUM_CLASSES)


def kernel(target, pred):
    del pred
    return _sc_call(target)
